# baseline (speedup 1.0000x reference)
import jax
import jax.numpy as jnp
from jax.experimental import pallas as pl

_NUM_CHANNELS = 5
_GRID = 10


def _channel_mean(v):
    acc = v
    for _ in range(_NUM_CHANNELS - 1):
        acc = acc + v
    return acc * jnp.float32(1.0 / _NUM_CHANNELS)


def _mean_kernel(x_ref, e_ref, u_ref, xo_ref, eo_ref, uo_ref):
    xo_ref[...] = _channel_mean(x_ref[...])
    eo_ref[...] = _channel_mean(e_ref[...])

    @pl.when(pl.program_id(0) == 0)
    def _():
        uo_ref[...] = _channel_mean(u_ref[...])


def kernel(x, edge_index, edge_attr, u, batch):
    del edge_index, batch
    n, d = x.shape
    e, de = edge_attr.shape
    ef = edge_attr.reshape(e * de // 128, 128)
    xb = n // _GRID
    eb = ef.shape[0] // _GRID

    x_m, e_m, u_m = pl.pallas_call(
        _mean_kernel,
        grid=(_GRID,),
        in_specs=[
            pl.BlockSpec((xb, d), lambda i: (i, 0)),
            pl.BlockSpec((eb, 128), lambda i: (i, 0)),
            pl.BlockSpec((1, d), lambda i: (0, 0)),
        ],
        out_specs=[
            pl.BlockSpec((xb, d), lambda i: (i, 0)),
            pl.BlockSpec((eb, 128), lambda i: (i, 0)),
            pl.BlockSpec((1, d), lambda i: (0, 0)),
        ],
        out_shape=[
            jax.ShapeDtypeStruct((n, d), x.dtype),
            jax.ShapeDtypeStruct(ef.shape, edge_attr.dtype),
            jax.ShapeDtypeStruct((1, d), u.dtype),
        ],
    )(x, ef, u)

    return (
        x_m[:, :, None],
        e_m.reshape(e, de)[:, :, None],
        u_m[:, :, None],
    )


# speedup vs baseline: 1.1980x; 1.1980x over previous
import jax
import jax.numpy as jnp
from jax.experimental import pallas as pl

_NUM_CHANNELS = 5
_GRID = 25


def _channel_mean(v):
    acc = v
    for _ in range(_NUM_CHANNELS - 1):
        acc = acc + v
    return acc * jnp.float32(1.0 / _NUM_CHANNELS)


def _mean_kernel(x_ref, e_ref, u_ref, xo_ref, eo_ref, uo_ref):
    xo_ref[...] = _channel_mean(x_ref[...])
    eo_ref[...] = _channel_mean(e_ref[...])

    @pl.when(pl.program_id(0) == 0)
    def _():
        uo_ref[...] = _channel_mean(u_ref[...])


def kernel(x, edge_index, edge_attr, u, batch):
    del edge_index, batch
    n, d = x.shape
    e, de = edge_attr.shape
    xb = n // _GRID
    eb = e // _GRID

    x_m, e_m, u_m = pl.pallas_call(
        _mean_kernel,
        grid=(_GRID,),
        in_specs=[
            pl.BlockSpec((xb, d), lambda i: (i, 0)),
            pl.BlockSpec((eb, de), lambda i: (i, 0)),
            pl.BlockSpec((1, d), lambda i: (0, 0)),
        ],
        out_specs=[
            pl.BlockSpec((xb, d), lambda i: (i, 0)),
            pl.BlockSpec((eb, de), lambda i: (i, 0)),
            pl.BlockSpec((1, d), lambda i: (0, 0)),
        ],
        out_shape=[
            jax.ShapeDtypeStruct((n, d), x.dtype),
            jax.ShapeDtypeStruct((e, de), edge_attr.dtype),
            jax.ShapeDtypeStruct((1, d), u.dtype),
        ],
    )(x, edge_attr, u)

    return (
        x_m[:, :, None],
        e_m[:, :, None],
        u_m[:, :, None],
    )


# speedup vs baseline: 7.7972x; 6.5086x over previous
import jax
import jax.numpy as jnp
from jax.experimental import pallas as pl
from jax.experimental.pallas import tpu as pltpu

_NUM_CHANNELS = 5
_MEAN_SCALE = float(_NUM_CHANNELS) * (1.0 / _NUM_CHANNELS)

_EG = 20


def _xu_kernel(x_ref, u_ref, xo_ref, uo_ref):
    xo_ref[...] = x_ref[...] * _MEAN_SCALE

    @pl.when(pl.program_id(0) == 0)
    def _():
        uo_ref[...] = u_ref[...] * _MEAN_SCALE


def _edge_kernel(e_ref, out_ref, scratch, sems):
    g = pl.program_id(0)
    j = pl.program_id(1)
    eg = pl.num_programs(1)
    ng = pl.num_programs(0)
    s = g * eg + j
    p = jax.lax.rem(s, 2)
    w = e_ref.shape[1]
    ecols = eg * w

    def row_copy(slot, gg, jj, k):
        base = (8 * gg + k) * ecols + jj * w
        return pltpu.make_async_copy(
            scratch.at[slot, k],
            out_ref.at[0, pl.ds(base, w)],
            sems.at[slot, k],
        )

    @pl.when(s >= 2)
    def _():
        g2 = (s - 2) // eg
        j2 = jax.lax.rem(s - 2, eg)
        for k in range(8):
            row_copy(p, g2, j2, k).wait()

    scratch[p] = e_ref[...] * _MEAN_SCALE
    for k in range(8):
        row_copy(p, g, j, k).start()

    @pl.when(s == ng * eg - 1)
    def _():
        @pl.when(s >= 1)
        def _():
            g1 = (s - 1) // eg
            j1 = jax.lax.rem(s - 1, eg)
            for k in range(8):
                row_copy(1 - p, g1, j1, k).wait()

        for k in range(8):
            row_copy(p, g, j, k).wait()


def kernel(x, edge_index, edge_attr, u, batch):
    del edge_index, batch
    n, d = x.shape
    e, de = edge_attr.shape

    xg = 10
    x_m, u_m = pl.pallas_call(
        _xu_kernel,
        grid=(xg,),
        in_specs=[
            pl.BlockSpec((n // xg, d), lambda i: (i, 0)),
            pl.BlockSpec((1, d), lambda i: (0, 0)),
        ],
        out_specs=[
            pl.BlockSpec((n // xg, d), lambda i: (i, 0)),
            pl.BlockSpec((1, d), lambda i: (0, 0)),
        ],
        out_shape=[
            jax.ShapeDtypeStruct((n, d), x.dtype),
            jax.ShapeDtypeStruct((1, d), u.dtype),
        ],
    )(x, u)

    et = jnp.transpose(edge_attr)
    w = e // _EG
    e_m = pl.pallas_call(
        _edge_kernel,
        grid=(de // 8, _EG),
        in_specs=[pl.BlockSpec((8, w), lambda g, j: (g, j))],
        out_specs=pl.BlockSpec(memory_space=pl.ANY),
        out_shape=jax.ShapeDtypeStruct((1, de * e), edge_attr.dtype),
        scratch_shapes=[
            pltpu.VMEM((2, 8, w), edge_attr.dtype),
            pltpu.SemaphoreType.DMA((2, 8)),
        ],
    )(et)

    return (
        x_m[:, :, None],
        e_m.reshape(de, e, 1).transpose(1, 0, 2),
        u_m[:, :, None],
    )
